# baseline (speedup 1.0000x reference)
import functools

import jax
import jax.numpy as jnp
from jax import lax
from jax.experimental import pallas as pl
from jax.experimental.pallas import tpu as pltpu
from jax.experimental.pallas import tpu_sc as plsc

N = 10000
D = 128
NEIG_S = 16
NEIG_K = 16


def _sample_neighbors(edge_index, k, key, num_nodes):
    row, col = edge_index[0], edge_index[1]
    order = jnp.argsort(row)
    row_s = row[order]
    col_s = col[order]
    nodes = jnp.arange(num_nodes)
    starts = jnp.searchsorted(row_s, nodes, side='left')
    ends = jnp.searchsorted(row_s, nodes, side='right')
    deg = ends - starts
    r = jax.random.randint(key, (num_nodes, k), 0, 1 << 30)
    pos = starts[:, None] + (r % jnp.maximum(deg, 1)[:, None])
    pos = jnp.clip(pos, 0, row_s.shape[0] - 1)
    return jnp.where(deg[:, None] > 0, col_s[pos], 0)


def _mm_kernel(a_ref, b_ref, o_ref):
    o_ref[...] = jax.lax.dot_general(
        a_ref[...], b_ref[...], (((1,), (0,)), ((), ())),
        precision=jax.lax.Precision.DEFAULT, preferred_element_type=jnp.float32)


def _matmul(a, b):
    m, k = a.shape
    _, n = b.shape
    blk = 2000
    return pl.pallas_call(
        _mm_kernel,
        grid=(m // blk,),
        in_specs=[pl.BlockSpec((blk, k), lambda i: (i, 0)),
                  pl.BlockSpec((k, n), lambda i: (0, 0))],
        out_specs=pl.BlockSpec((blk, n), lambda i: (i, 0)),
        out_shape=jax.ShapeDtypeStruct((m, n), jnp.float32),
    )(a, b)


def _topk_iter_full(nd, cols, k, n):
    outs = []
    for _ in range(k):
        m = jnp.max(nd, axis=1)
        sel = jnp.min(jnp.where(nd == m[:, None], cols, n), axis=1)
        outs.append(sel[:, None])
        nd = jnp.where(cols == sel[:, None], -jnp.inf, nd)
    return jnp.concatenate(outs, axis=1)


def _ce(av, ai, bv, bi):
    take_a = (av > bv) | ((av == bv) & (ai < bi))
    hv = jnp.where(take_a, av, bv)
    hi = jnp.where(take_a, ai, bi)
    lv = jnp.where(take_a, bv, av)
    li = jnp.where(take_a, bi, ai)
    return hv, hi, lv, li


def _merge4_top4(a, b):
    av, ai = a
    bv, bi = b
    m = [None] * 4
    for t in range(4):
        hv, hi, _, _ = _ce(av[t], ai[t], bv[3 - t], bi[3 - t])
        m[t] = (hv, hi)
    for (p, q) in ((0, 2), (1, 3), (0, 1), (2, 3)):
        hv, hi, lv, li = _ce(m[p][0], m[p][1], m[q][0], m[q][1])
        m[p] = (hv, hi)
        m[q] = (lv, li)
    return [x[0] for x in m], [x[1] for x in m]


def _knn_body(blk_r, k, xb_ref, xa_ref, idx_ref):
    i = pl.program_id(0)
    xb = xb_ref[...]
    xa = xa_ref[...]
    g = jax.lax.dot_general(xb, xa, (((1,), (1,)), ((), ())),
                            precision=jax.lax.Precision.DEFAULT,
                            preferred_element_type=jnp.float32)
    sqb = jnp.sum(xb * xb, axis=1)
    sqa = jnp.sum(xa * xa, axis=1)
    nd = -((sqb[:, None] - 2.0 * g) + sqa[None, :])
    n = nd.shape[1]
    cols = jax.lax.broadcasted_iota(jnp.int32, (blk_r, n), 1)
    rows = i * blk_r + jax.lax.broadcasted_iota(jnp.int32, (blk_r, n), 0)
    nd = jnp.where(cols == rows, -jnp.inf, nd)

    w = n // 4
    quads = [(nd[:, j * w:(j + 1) * w], cols[:, j * w:(j + 1) * w]) for j in range(4)]
    h1v, h1i, l1v, l1i = _ce(quads[0][0], quads[0][1], quads[1][0], quads[1][1])
    h2v, h2i, l2v, l2i = _ce(quads[2][0], quads[2][1], quads[3][0], quads[3][1])
    x1v, x1i, y1v, y1i = _ce(h1v, h1i, h2v, h2i)
    x2v, x2i, y2v, y2i = _ce(l1v, l1i, l2v, l2i)
    midv, midi, lowv, lowi = _ce(y1v, y1i, x2v, x2i)
    lvl = ([x1v, midv, lowv, y2v], [x1i, midi, lowi, y2i])
    while lvl[0][0].shape[1] > n // 16:
        half = lvl[0][0].shape[1] // 2
        a = ([v[:, :half] for v in lvl[0]], [ix[:, :half] for ix in lvl[1]])
        b = ([v[:, half:] for v in lvl[0]], [ix[:, half:] for ix in lvl[1]])
        lvl = _merge4_top4(a, b)
    gv, gi = lvl

    hv, hi = gv[0], gi[0]
    dep = jnp.zeros(hv.shape, jnp.int32)
    neg = jnp.full(hv.shape, -jnp.inf, jnp.float32)
    big = jnp.full(hi.shape, n, jnp.int32)
    outs = []
    for _ in range(k):
        m = jnp.max(hv, axis=1)
        sel = jnp.min(jnp.where(hv == m[:, None], hi, n), axis=1)
        outs.append(sel[:, None])
        onehot = hi == sel[:, None]
        dep = dep + onehot.astype(jnp.int32)
        nxtv = jnp.where(dep == 1, gv[1], jnp.where(dep == 2, gv[2],
                         jnp.where(dep == 3, gv[3], neg)))
        nxti = jnp.where(dep == 1, gi[1], jnp.where(dep == 2, gi[2],
                         jnp.where(dep == 3, gi[3], big)))
        hv = jnp.where(onehot, nxtv, hv)
        hi = jnp.where(onehot, nxti, hi)
    fast = jnp.concatenate(outs, axis=1)
    idx_ref[...] = fast
    overflow = jnp.any(dep >= 4)

    @pl.when(overflow)
    def _():
        idx_ref[...] = _topk_iter_full(nd, cols, k, n)


def _knn_topk(xw, k):
    n, d = xw.shape
    blk_r = 200
    return pl.pallas_call(
        functools.partial(_knn_body, blk_r, k),
        grid=(n // blk_r,),
        in_specs=[pl.BlockSpec((blk_r, d), lambda i: (i, 0)),
                  pl.BlockSpec((n, d), lambda i: (0, 0))],
        out_specs=pl.BlockSpec((blk_r, k), lambda i: (i, 0)),
        out_shape=jax.ShapeDtypeStruct((n, k), jnp.int32),
    )(xw, xw)


_SC_CHUNK = 200


def _sc_gather(table, idx_flat):
    b = idx_flat.shape[0]
    d = table.shape[1]
    info = plsc.get_sparse_core_info()
    nw = info.num_cores * info.num_subcores
    b_per_w = b // nw
    n_chunks = b_per_w // _SC_CHUNK
    mesh = plsc.VectorSubcoreMesh(core_axis_name="c", subcore_axis_name="s")

    @functools.partial(
        pl.kernel, mesh=mesh,
        out_type=jax.ShapeDtypeStruct((b, d), jnp.float32),
        scratch_types=[
            pltpu.VMEM((_SC_CHUNK,), jnp.int32),
            pltpu.VMEM((_SC_CHUNK, d), jnp.float32),
            pltpu.SemaphoreType.DMA,
        ],
    )
    def gather_kernel(table_hbm, idx_hbm, out_hbm, idx_v, rows_v, sem):
        wid = lax.axis_index("s") * info.num_cores + lax.axis_index("c")
        base = wid * b_per_w
        for c in range(n_chunks):
            off = base + c * _SC_CHUNK
            pltpu.sync_copy(idx_hbm.at[pl.ds(off, _SC_CHUNK)], idx_v)
            pltpu.async_copy(table_hbm.at[idx_v], rows_v, sem).wait()
            pltpu.sync_copy(rows_v, out_hbm.at[pl.ds(off, _SC_CHUNK)])

    return gather_kernel(table, idx_flat)


def _conv_block(blk_n, k, r3, wt_ref, bkk0_ref, bkk1_ref, wk1_ref, bk1_ref):
    d = D
    acc0 = jnp.zeros((blk_n, d), jnp.float32)
    acc1 = jnp.zeros((blk_n, d), jnp.float32)
    for t in range(k):
        rt = r3[:, t, :]
        acc0 = acc0 + rt * wt_ref[t, :][None, :]
        acc1 = acc1 + rt * wt_ref[k + t, :][None, :]
    conv0 = acc0 + bkk0_ref[...]
    conv1 = acc1 + bkk1_ref[...]
    glob = jnp.maximum(conv0, conv1)
    mparts = [jnp.max(glob[:, 8 * i:8 * i + 8], axis=1, keepdims=True)
              for i in range(16)]
    m = jnp.concatenate(mparts, axis=1)
    lane = jax.lax.broadcasted_iota(jnp.int32, (16, d), 1)
    grp = jax.lax.broadcasted_iota(jnp.int32, (16, d), 0)
    e128 = (lane // 8 == grp).astype(jnp.float32)
    mb = jax.lax.dot_general(m, e128, (((1,), (0,)), ((), ())),
                             precision=jax.lax.Precision.HIGHEST,
                             preferred_element_type=jnp.float32)
    e0 = jnp.exp(conv0 - mb)
    e1 = jnp.exp(conv1 - mb)
    s = jax.lax.dot_general(e0 + e1, e128, (((1,), (1,)), ((), ())),
                            precision=jax.lax.Precision.HIGHEST,
                            preferred_element_type=jnp.float32)
    sb = jax.lax.dot_general(s, e128, (((1,), (0,)), ((), ())),
                             precision=jax.lax.Precision.HIGHEST,
                             preferred_element_type=jnp.float32)
    mult0 = e0 / sb
    mult1 = e1 / sb
    wk1b = jax.lax.dot_general(wk1_ref[...], e128, (((1,), (0,)), ((), ())),
                               precision=jax.lax.Precision.HIGHEST,
                               preferred_element_type=jnp.float32)
    lane8 = jax.lax.broadcasted_iota(jnp.int32, (d, 8), 0)
    p8 = jax.lax.broadcasted_iota(jnp.int32, (d, 8), 1)
    pmat = (lane8 % 8 == p8).astype(jnp.float32)
    coef0 = jax.lax.dot_general(mult0 * wk1b, pmat, (((1,), (0,)), ((), ())),
                                precision=jax.lax.Precision.HIGHEST,
                                preferred_element_type=jnp.float32)
    coef1 = jax.lax.dot_general(mult1 * wk1b, pmat, (((1,), (0,)), ((), ())),
                                precision=jax.lax.Precision.HIGHEST,
                                preferred_element_type=jnp.float32)
    pooled = jnp.zeros((blk_n, d), jnp.float32)
    for p in range(8):
        pooled = pooled + coef0[:, p:p + 1] * r3[:, 2 * p, :]
        pooled = pooled + coef1[:, p:p + 1] * r3[:, 2 * p + 1, :]
    return pooled + bk1_ref[0, 0]


def _conv2_body(blk_n, k, rs_ref, rk_ref, wts_ref, bkk0s_ref, bkk1s_ref,
                wk1s_ref, bk1s_ref, wtk_ref, bkk0k_ref, bkk1k_ref,
                wk1k_ref, bk1k_ref, bias_ref, o_ref):
    d = D
    rs3 = rs_ref[...].reshape(blk_n, k, d)
    rk3 = rk_ref[...].reshape(blk_n, k, d)
    ps = _conv_block(blk_n, k, rs3, wts_ref, bkk0s_ref, bkk1s_ref, wk1s_ref, bk1s_ref)
    pk = _conv_block(blk_n, k, rk3, wtk_ref, bkk0k_ref, bkk1k_ref, wk1k_ref, bk1k_ref)
    o_ref[...] = ps + pk + bias_ref[...]


def _conv_weights(Wkk, bkk, Wk1, k):
    d = D
    wt = jnp.transpose(Wkk.reshape(d, 2, k), (1, 2, 0)).reshape(2 * k, d)
    return wt, bkk[0::2].reshape(1, d), bkk[1::2].reshape(1, d), Wk1[0, :, 0].reshape(1, k)


def _conv_combined(region_all, ws, wk, bias, k):
    n = region_all.shape[0] // (2 * k)
    d = region_all.shape[1]
    blk_n = 400
    nblk = n // blk_n
    w_spec = [
        pl.BlockSpec((2 * k, d), lambda i: (0, 0)),
        pl.BlockSpec((1, d), lambda i: (0, 0)),
        pl.BlockSpec((1, d), lambda i: (0, 0)),
        pl.BlockSpec((1, k), lambda i: (0, 0)),
        pl.BlockSpec((1, 1), lambda i: (0, 0)),
    ]
    return pl.pallas_call(
        functools.partial(_conv2_body, blk_n, k),
        grid=(nblk,),
        in_specs=[
            pl.BlockSpec((blk_n * k, d), lambda i: (i, 0)),
            pl.BlockSpec((blk_n * k, d), lambda i: (i + nblk, 0)),
        ] + w_spec + w_spec + [pl.BlockSpec((1, d), lambda i: (0, 0))],
        out_specs=pl.BlockSpec((blk_n, d), lambda i: (i, 0)),
        out_shape=jax.ShapeDtypeStruct((n, d), jnp.float32),
    )(region_all, region_all, *ws, *wk, bias.reshape(1, d))


def kernel(x, edge_index, weight, bias, convKK_s_w, convKK_s_b, convK1_s_w, convK1_s_b,
           convKK_k_w, convKK_k_b, convK1_k_w, convK1_k_b, att_w1, att_b1, att_w2, att_b2):
    n = x.shape[0]
    key = jax.random.key(42)
    k1, k2 = jax.random.split(key)
    xw = _matmul(x, weight)
    neigh_s = _sample_neighbors(edge_index, NEIG_S, k1, n)
    knn_idx = _knn_topk(xw, NEIG_K)
    sel = jax.random.randint(k2, (n, NEIG_K), 0, NEIG_K)
    neigh_k = jnp.take_along_axis(knn_idx, sel, axis=1)
    idx_all = jnp.concatenate([neigh_s.reshape(-1), neigh_k.reshape(-1)])
    region_all = _sc_gather(xw, idx_all)
    ws = _conv_weights(convKK_s_w, convKK_s_b, convK1_s_w, NEIG_S) + (convK1_s_b.reshape(1, 1),)
    wk = _conv_weights(convKK_k_w, convKK_k_b, convK1_k_w, NEIG_K) + (convK1_k_b.reshape(1, 1),)
    return _conv_combined(region_all, ws, wk, bias, NEIG_K)


# speedup vs baseline: 1.0224x; 1.0224x over previous
import functools

import jax
import jax.numpy as jnp
from jax import lax
from jax.experimental import pallas as pl
from jax.experimental.pallas import tpu as pltpu
from jax.experimental.pallas import tpu_sc as plsc

N = 10000
D = 128
NEIG_S = 16
NEIG_K = 16


def _sample_neighbors(edge_index, k, key, num_nodes):
    row, col = edge_index[0], edge_index[1]
    order = jnp.argsort(row)
    row_s = row[order]
    col_s = col[order]
    nodes = jnp.arange(num_nodes)
    starts = jnp.searchsorted(row_s, nodes, side='left')
    ends = jnp.searchsorted(row_s, nodes, side='right')
    deg = ends - starts
    r = jax.random.randint(key, (num_nodes, k), 0, 1 << 30)
    pos = starts[:, None] + (r % jnp.maximum(deg, 1)[:, None])
    pos = jnp.clip(pos, 0, row_s.shape[0] - 1)
    return jnp.where(deg[:, None] > 0, col_s[pos], 0)


def _mm_kernel(a_ref, b_ref, o_ref):
    o_ref[...] = jax.lax.dot_general(
        a_ref[...], b_ref[...], (((1,), (0,)), ((), ())),
        precision=jax.lax.Precision.DEFAULT, preferred_element_type=jnp.float32)


def _matmul(a, b):
    m, k = a.shape
    _, n = b.shape
    blk = 2000
    return pl.pallas_call(
        _mm_kernel,
        grid=(m // blk,),
        in_specs=[pl.BlockSpec((blk, k), lambda i: (i, 0)),
                  pl.BlockSpec((k, n), lambda i: (0, 0))],
        out_specs=pl.BlockSpec((blk, n), lambda i: (i, 0)),
        out_shape=jax.ShapeDtypeStruct((m, n), jnp.float32),
    )(a, b)


def _topk_iter_full(nd, cols, k, n):
    outs = []
    for _ in range(k):
        m = jnp.max(nd, axis=1)
        sel = jnp.min(jnp.where(nd == m[:, None], cols, n), axis=1)
        outs.append(sel[:, None])
        nd = jnp.where(cols == sel[:, None], -jnp.inf, nd)
    return jnp.concatenate(outs, axis=1)


def _ce(av, ai, bv, bi):
    take_a = (av > bv) | ((av == bv) & (ai < bi))
    hv = jnp.where(take_a, av, bv)
    hi = jnp.where(take_a, ai, bi)
    lv = jnp.where(take_a, bv, av)
    li = jnp.where(take_a, bi, ai)
    return hv, hi, lv, li


def _merge4_top4(a, b):
    av, ai = a
    bv, bi = b
    m = [None] * 4
    for t in range(4):
        hv, hi, _, _ = _ce(av[t], ai[t], bv[3 - t], bi[3 - t])
        m[t] = (hv, hi)
    for (p, q) in ((0, 2), (1, 3), (0, 1), (2, 3)):
        hv, hi, lv, li = _ce(m[p][0], m[p][1], m[q][0], m[q][1])
        m[p] = (hv, hi)
        m[q] = (lv, li)
    return [x[0] for x in m], [x[1] for x in m]


def _knn_body(blk_r, k, xb_ref, xa_ref, idx_ref):
    i = pl.program_id(0)
    xb = xb_ref[...]
    xa = xa_ref[...]
    g = jax.lax.dot_general(xb, xa, (((1,), (1,)), ((), ())),
                            precision=jax.lax.Precision.DEFAULT,
                            preferred_element_type=jnp.float32)
    sqb = jnp.sum(xb * xb, axis=1)
    sqa = jnp.sum(xa * xa, axis=1)
    nd = -((sqb[:, None] - 2.0 * g) + sqa[None, :])
    n = nd.shape[1]
    cols = jax.lax.broadcasted_iota(jnp.int32, (blk_r, n), 1)
    rows = i * blk_r + jax.lax.broadcasted_iota(jnp.int32, (blk_r, n), 0)
    nd = jnp.where(cols == rows, -jnp.inf, nd)

    w = n // 4
    quads = [(nd[:, j * w:(j + 1) * w], cols[:, j * w:(j + 1) * w]) for j in range(4)]
    h1v, h1i, l1v, l1i = _ce(quads[0][0], quads[0][1], quads[1][0], quads[1][1])
    h2v, h2i, l2v, l2i = _ce(quads[2][0], quads[2][1], quads[3][0], quads[3][1])
    x1v, x1i, y1v, y1i = _ce(h1v, h1i, h2v, h2i)
    x2v, x2i, y2v, y2i = _ce(l1v, l1i, l2v, l2i)
    midv, midi, lowv, lowi = _ce(y1v, y1i, x2v, x2i)
    lvl = ([x1v, midv, lowv, y2v], [x1i, midi, lowi, y2i])
    while lvl[0][0].shape[1] > n // 16:
        half = lvl[0][0].shape[1] // 2
        a = ([v[:, :half] for v in lvl[0]], [ix[:, :half] for ix in lvl[1]])
        b = ([v[:, half:] for v in lvl[0]], [ix[:, half:] for ix in lvl[1]])
        lvl = _merge4_top4(a, b)
    gv, gi = lvl

    hv, hi = gv[0], gi[0]
    dep = jnp.zeros(hv.shape, jnp.int32)
    neg = jnp.full(hv.shape, -jnp.inf, jnp.float32)
    big = jnp.full(hi.shape, n, jnp.int32)
    outs = []
    for _ in range(k):
        m = jnp.max(hv, axis=1)
        sel = jnp.min(jnp.where(hv == m[:, None], hi, n), axis=1)
        outs.append(sel[:, None])
        onehot = hi == sel[:, None]
        dep = dep + onehot.astype(jnp.int32)
        nxtv = jnp.where(dep == 1, gv[1], jnp.where(dep == 2, gv[2],
                         jnp.where(dep == 3, gv[3], neg)))
        nxti = jnp.where(dep == 1, gi[1], jnp.where(dep == 2, gi[2],
                         jnp.where(dep == 3, gi[3], big)))
        hv = jnp.where(onehot, nxtv, hv)
        hi = jnp.where(onehot, nxti, hi)
    fast = jnp.concatenate(outs, axis=1)
    idx_ref[...] = fast
    overflow = jnp.any(dep >= 4)

    @pl.when(overflow)
    def _():
        idx_ref[...] = _topk_iter_full(nd, cols, k, n)


def _knn_topk(xw, k):
    n, d = xw.shape
    blk_r = 200
    return pl.pallas_call(
        functools.partial(_knn_body, blk_r, k),
        grid=(n // blk_r,),
        in_specs=[pl.BlockSpec((blk_r, d), lambda i: (i, 0)),
                  pl.BlockSpec((n, d), lambda i: (0, 0))],
        out_specs=pl.BlockSpec((blk_r, k), lambda i: (i, 0)),
        out_shape=jax.ShapeDtypeStruct((n, k), jnp.int32),
    )(xw, xw)


_SC_CHUNK = 200


def _sc_gather(table, idx_flat):
    b = idx_flat.shape[0]
    d = table.shape[1]
    info = plsc.get_sparse_core_info()
    nw = info.num_cores * info.num_subcores
    b_per_w = b // nw
    n_chunks = b_per_w // _SC_CHUNK
    mesh = plsc.VectorSubcoreMesh(core_axis_name="c", subcore_axis_name="s")

    @functools.partial(
        pl.kernel, mesh=mesh,
        out_type=jax.ShapeDtypeStruct((b, d), jnp.float32),
        scratch_types=[
            pltpu.VMEM((_SC_CHUNK,), jnp.int32),
            pltpu.VMEM((_SC_CHUNK, d), jnp.float32),
            pltpu.SemaphoreType.DMA,
        ],
    )
    def gather_kernel(table_hbm, idx_hbm, out_hbm, idx_v, rows_v, sem):
        wid = lax.axis_index("s") * info.num_cores + lax.axis_index("c")
        base = wid * b_per_w
        for c in range(n_chunks):
            off = base + c * _SC_CHUNK
            pltpu.sync_copy(idx_hbm.at[pl.ds(off, _SC_CHUNK)], idx_v)
            pltpu.async_copy(table_hbm.at[idx_v], rows_v, sem).wait()
            pltpu.sync_copy(rows_v, out_hbm.at[pl.ds(off, _SC_CHUNK)])

    return gather_kernel(table, idx_flat)


def _conv_block(blk_n, k, r3, wt_ref, bkk0_ref, bkk1_ref, wk1_ref, bk1_ref):
    d = D
    acc0 = jnp.zeros((blk_n, d), jnp.float32)
    acc1 = jnp.zeros((blk_n, d), jnp.float32)
    for t in range(k):
        rt = r3[:, t, :]
        acc0 = acc0 + rt * wt_ref[t, :][None, :]
        acc1 = acc1 + rt * wt_ref[k + t, :][None, :]
    conv0 = acc0 + bkk0_ref[...]
    conv1 = acc1 + bkk1_ref[...]
    glob = jnp.maximum(conv0, conv1)
    mparts = [jnp.max(glob[:, 8 * i:8 * i + 8], axis=1, keepdims=True)
              for i in range(16)]
    m = jnp.concatenate(mparts, axis=1)
    lane = jax.lax.broadcasted_iota(jnp.int32, (16, d), 1)
    grp = jax.lax.broadcasted_iota(jnp.int32, (16, d), 0)
    e128 = (lane // 8 == grp).astype(jnp.float32)
    mb = jax.lax.dot_general(m, e128, (((1,), (0,)), ((), ())),
                             precision=jax.lax.Precision.HIGHEST,
                             preferred_element_type=jnp.float32)
    e0 = jnp.exp(conv0 - mb)
    e1 = jnp.exp(conv1 - mb)
    s = jax.lax.dot_general(e0 + e1, e128, (((1,), (1,)), ((), ())),
                            precision=jax.lax.Precision.HIGHEST,
                            preferred_element_type=jnp.float32)
    sb = jax.lax.dot_general(s, e128, (((1,), (0,)), ((), ())),
                             precision=jax.lax.Precision.HIGHEST,
                             preferred_element_type=jnp.float32)
    mult0 = e0 / sb
    mult1 = e1 / sb
    wk1b = jax.lax.dot_general(wk1_ref[...], e128, (((1,), (0,)), ((), ())),
                               precision=jax.lax.Precision.HIGHEST,
                               preferred_element_type=jnp.float32)
    lane8 = jax.lax.broadcasted_iota(jnp.int32, (d, 8), 0)
    p8 = jax.lax.broadcasted_iota(jnp.int32, (d, 8), 1)
    pmat = (lane8 % 8 == p8).astype(jnp.float32)
    coef0 = jax.lax.dot_general(mult0 * wk1b, pmat, (((1,), (0,)), ((), ())),
                                precision=jax.lax.Precision.HIGHEST,
                                preferred_element_type=jnp.float32)
    coef1 = jax.lax.dot_general(mult1 * wk1b, pmat, (((1,), (0,)), ((), ())),
                                precision=jax.lax.Precision.HIGHEST,
                                preferred_element_type=jnp.float32)
    pooled = jnp.zeros((blk_n, d), jnp.float32)
    for p in range(8):
        pooled = pooled + coef0[:, p:p + 1] * r3[:, 2 * p, :]
        pooled = pooled + coef1[:, p:p + 1] * r3[:, 2 * p + 1, :]
    return pooled + bk1_ref[0, 0]


def _conv_body(blk_n, k, r_ref, wt_ref, bkk0_ref, bkk1_ref, wk1_ref, bk1_ref, o_ref):
    r3 = r_ref[...].reshape(blk_n, k, D)
    o_ref[...] = _conv_block(blk_n, k, r3, wt_ref, bkk0_ref, bkk1_ref, wk1_ref, bk1_ref)


def _conv_pallas(region, Wkk, bkk, Wk1, bk1, k):
    n = region.shape[0] // k
    d = region.shape[1]
    wt = jnp.transpose(Wkk.reshape(d, 2, k), (1, 2, 0)).reshape(2 * k, d)
    bkk0 = bkk[0::2].reshape(1, d)
    bkk1 = bkk[1::2].reshape(1, d)
    wk1 = Wk1[0, :, 0].reshape(1, k)
    blk_n = 400
    return pl.pallas_call(
        functools.partial(_conv_body, blk_n, k),
        grid=(n // blk_n,),
        in_specs=[
            pl.BlockSpec((blk_n * k, d), lambda i: (i, 0)),
            pl.BlockSpec((2 * k, d), lambda i: (0, 0)),
            pl.BlockSpec((1, d), lambda i: (0, 0)),
            pl.BlockSpec((1, d), lambda i: (0, 0)),
            pl.BlockSpec((1, k), lambda i: (0, 0)),
            pl.BlockSpec((1, 1), lambda i: (0, 0)),
        ],
        out_specs=pl.BlockSpec((blk_n, d), lambda i: (i, 0)),
        out_shape=jax.ShapeDtypeStruct((n, d), jnp.float32),
    )(region, wt, bkk0, bkk1, wk1, bk1.reshape(1, 1))


def _att_kernel(xs_ref, xk_ref, bias_ref, out_ref):
    out_ref[...] = xs_ref[...] + xk_ref[...] + bias_ref[...]


def _attention(x_s, x_k, bias):
    n, d = x_s.shape
    blk = 2000
    return pl.pallas_call(
        _att_kernel,
        grid=(n // blk,),
        in_specs=[
            pl.BlockSpec((blk, d), lambda i: (i, 0)),
            pl.BlockSpec((blk, d), lambda i: (i, 0)),
            pl.BlockSpec((1, d), lambda i: (0, 0)),
        ],
        out_specs=pl.BlockSpec((blk, d), lambda i: (i, 0)),
        out_shape=jax.ShapeDtypeStruct((n, d), jnp.float32),
    )(x_s, x_k, bias.reshape(1, -1))


def kernel(x, edge_index, weight, bias, convKK_s_w, convKK_s_b, convK1_s_w, convK1_s_b,
           convKK_k_w, convKK_k_b, convK1_k_w, convK1_k_b, att_w1, att_b1, att_w2, att_b2):
    n = x.shape[0]
    key = jax.random.key(42)
    k1, k2 = jax.random.split(key)
    xw = _matmul(x, weight)
    neigh_s = _sample_neighbors(edge_index, NEIG_S, k1, n)
    knn_idx = _knn_topk(xw, NEIG_K)
    sel = jax.random.randint(k2, (n, NEIG_K), 0, NEIG_K)
    neigh_k = jnp.take_along_axis(knn_idx, sel, axis=1)
    region_s = _sc_gather(xw, neigh_s.reshape(-1))
    region_k = _sc_gather(xw, neigh_k.reshape(-1))
    x_s = _conv_pallas(region_s, convKK_s_w, convKK_s_b, convK1_s_w, convK1_s_b, NEIG_S)
    x_k = _conv_pallas(region_k, convKK_k_w, convKK_k_b, convK1_k_w, convK1_k_b, NEIG_K)
    return _attention(x_s, x_k, bias)
